# baseline (speedup 1.0000x reference)
import functools

import jax
import jax.numpy as jnp
from jax import lax
from jax.experimental import pallas as pl
from jax.experimental.pallas import tpu as pltpu
from jax.experimental.pallas import tpu_sc as plsc

_B, _K, _D = 128, 32, 2048
_NC, _NS = 1, 16
_NW = _NC * _NS
_BPW = _B // _NW
_L = 16


@functools.lru_cache(maxsize=None)
def _build(len_dtype_name):
    len_dtype = jnp.dtype(len_dtype_name)
    mesh = plsc.VectorSubcoreMesh(
        core_axis_name="c", subcore_axis_name="s", num_cores=1)

    @functools.partial(
        pl.kernel,
        mesh=mesh,
        out_type=(
            jax.ShapeDtypeStruct((_B, _D), jnp.float32),
            jax.ShapeDtypeStruct((_B,), len_dtype),
        ),
        scratch_types=[
            pltpu.VMEM((_BPW, _D), jnp.float32),
            pltpu.VMEM((_B,), jnp.int32),
            pltpu.VMEM((_B,), len_dtype),
            pltpu.SemaphoreType.DMA,
            pltpu.SemaphoreType.DMA,
        ],
    )
    def top1(cand_hbm, lens_flat_hbm, out_hbm, olen_hbm,
             rows_v, idx_v, lens_v, sem, lsem):
        wid = lax.axis_index("s") * _NC + lax.axis_index("c")
        base = wid * _BPW
        row_cps = [
            pltpu.async_copy(
                cand_hbm.at[base + i, pl.ds(0, _D)],
                out_hbm.at[base + i],
                sem,
            )
            for i in range(_BPW)
        ]

        @pl.when(wid == 0)
        def _lengths():
            for i in range(_B // _L):
                idx_v[pl.ds(i * _L, _L)] = (
                    lax.iota(jnp.int32, _L) + (i * _L)) * _K
            pltpu.async_copy(lens_flat_hbm.at[idx_v], lens_v, lsem).wait()
            pltpu.sync_copy(lens_v, olen_hbm)

        for c in row_cps:
            c.wait()

    return top1


def kernel(candidates, lengths, batch, tgt_field, scores):
    cand_flat = candidates.reshape(_B, _K * _D)
    lens_flat = lengths.reshape(_B * _K)
    out, olens = _build(str(lens_flat.dtype))(cand_flat, lens_flat)
    return (out, olens, scores)


# speedup vs baseline: 1.5739x; 1.5739x over previous
import functools

import jax
import jax.numpy as jnp
from jax import lax
from jax.experimental import pallas as pl
from jax.experimental.pallas import tpu as pltpu
from jax.experimental.pallas import tpu_sc as plsc

_B, _K, _D = 128, 32, 2048
_NC, _NS = 1, 16
_NW = _NC * _NS
_BPW = _B // _NW
_L = 16


@functools.lru_cache(maxsize=None)
def _build(len_dtype_name):
    len_dtype = jnp.dtype(len_dtype_name)
    mesh = plsc.VectorSubcoreMesh(
        core_axis_name="c", subcore_axis_name="s", num_cores=1)

    @functools.partial(
        pl.kernel,
        mesh=mesh,
        out_type=(
            jax.ShapeDtypeStruct((_B, _D), jnp.float32),
            jax.ShapeDtypeStruct((_B,), len_dtype),
        ),
        scratch_types=[
            pltpu.VMEM((_BPW, _D), jnp.float32),
            pltpu.VMEM((_B,), jnp.int32),
            pltpu.VMEM((_B,), len_dtype),
            pltpu.SemaphoreType.DMA,
            pltpu.SemaphoreType.DMA,
        ],
    )
    def top1(cand_hbm, lens_flat_hbm, out_hbm, olen_hbm,
             rows_v, idx_v, lens_v, sem, lsem):
        wid = lax.axis_index("s") * _NC + lax.axis_index("c")
        base = wid * _BPW
        row_cps = [
            pltpu.async_copy(
                cand_hbm.at[base + i, pl.ds(0, _D)], rows_v.at[i], sem)
            for i in range(_BPW)
        ]

        @pl.when(wid == 0)
        def _lengths():
            for i in range(_B // _L):
                idx_v[pl.ds(i * _L, _L)] = (
                    lax.iota(jnp.int32, _L) + (i * _L)) * _K
            pltpu.async_copy(lens_flat_hbm.at[idx_v], lens_v, lsem).wait()
            pltpu.sync_copy(lens_v, olen_hbm)

        for c in row_cps:
            c.wait()
        pltpu.sync_copy(rows_v, out_hbm.at[pl.ds(base, _BPW)])

    return top1


def kernel(candidates, lengths, batch, tgt_field, scores):
    cand_flat = candidates.reshape(_B, _K * _D)
    lens_flat = lengths.reshape(_B * _K)
    out, olens = _build(str(lens_flat.dtype))(cand_flat, lens_flat)
    return (out, olens, scores)


# speedup vs baseline: 3.6719x; 2.3330x over previous
import functools

import jax
import jax.numpy as jnp
from jax import lax
from jax.experimental import pallas as pl
from jax.experimental.pallas import tpu as pltpu
from jax.experimental.pallas import tpu_sc as plsc

_B, _K, _D = 128, 32, 2048
_NC, _NS = 1, 16
_NW = _NC * _NS
_BPW = _B // _NW
_L = 16


@functools.lru_cache(maxsize=None)
def _build(len_dtype_name):
    len_dtype = jnp.dtype(len_dtype_name)
    mesh = plsc.VectorSubcoreMesh(
        core_axis_name="c", subcore_axis_name="s", num_cores=1)

    @functools.partial(
        pl.kernel,
        mesh=mesh,
        out_type=(
            jax.ShapeDtypeStruct((_B, _D), jnp.float32),
            jax.ShapeDtypeStruct((_B,), len_dtype),
        ),
        scratch_types=[
            pltpu.VMEM((_BPW, _D), jnp.float32),
            pltpu.VMEM((_B,), jnp.int32),
            pltpu.VMEM((_B,), len_dtype),
            pltpu.SemaphoreType.DMA,
            pltpu.SemaphoreType.DMA,
        ],
    )
    def top1(cand_hbm, lens_flat_hbm, out_hbm, olen_hbm,
             rows_v, idx_v, lens_v, sem, lsem):
        wid = lax.axis_index("s") * _NC + lax.axis_index("c")
        base = wid * _BPW
        row_cps = [
            pltpu.async_copy(
                cand_hbm.at[base + i, 0], rows_v.at[i], sem)
            for i in range(_BPW)
        ]

        @pl.when(wid == 0)
        def _lengths():
            for i in range(_B // _L):
                idx_v[pl.ds(i * _L, _L)] = (
                    lax.iota(jnp.int32, _L) + (i * _L)) * _K
            pltpu.async_copy(lens_flat_hbm.at[idx_v], lens_v, lsem).wait()
            pltpu.sync_copy(lens_v, olen_hbm)

        for c in row_cps:
            c.wait()
        pltpu.sync_copy(rows_v, out_hbm.at[pl.ds(base, _BPW)])

    return top1


def kernel(candidates, lengths, batch, tgt_field, scores):
    lens_flat = lengths.reshape(_B * _K)
    out, olens = _build(str(lens_flat.dtype))(candidates, lens_flat)
    return (out, olens, scores)
